# Initial kernel scaffold; baseline (speedup 1.0000x reference)
import functools

import jax
import jax.numpy as jnp
from jax import lax
from jax.experimental import pallas as pl
from jax.experimental.pallas import tpu as pltpu
from jax.experimental.pallas import tpu_sc as plsc

F_COUNT = 26
VOCAB = 100000
D = 32
BATCH = 4096
SEQ = 20

N_POS = BATCH * SEQ
NC, NS = 2, 16
NW = NC * NS
POS_PER_W = N_POS // NW
C = 32
CHUNKS = POS_PER_W // C
ROWS = C * F_COUNT
N_ROWS = N_POS * F_COUNT


def _sc_tokenize(ids_flat, tab_flat, offs_tiled, bias):
    mesh = plsc.VectorSubcoreMesh(core_axis_name="c", subcore_axis_name="s")

    @functools.partial(
        pl.kernel,
        mesh=mesh,
        out_type=jax.ShapeDtypeStruct((N_ROWS, D), jnp.float32),
        scratch_types=[
            pltpu.VMEM((ROWS,), jnp.int32),
            pltpu.VMEM((ROWS, D), jnp.float32),
            pltpu.VMEM((ROWS,), jnp.int32),
            pltpu.VMEM((F_COUNT, D), jnp.float32),
            pltpu.SemaphoreType.DMA,
        ],
    )
    def k(ids_hbm, tab_hbm, offs_hbm, bias_hbm, out_hbm,
          idx_v, rows_v, offs_v, bias_v, sem):
        wid = lax.axis_index("s") * NC + lax.axis_index("c")
        base = wid * POS_PER_W * F_COUNT
        pltpu.sync_copy(offs_hbm, offs_v)
        pltpu.sync_copy(bias_hbm, bias_v)

        @pl.loop(0, CHUNKS)
        def chunk(g):
            row0 = base + g * ROWS
            pltpu.sync_copy(ids_hbm.at[pl.ds(row0, ROWS)], idx_v)

            @pl.loop(0, ROWS // 16)
            def addoff(i):
                sl = pl.ds(i * 16, 16)
                idx_v[sl] = idx_v[sl] + offs_v[sl]

            pltpu.async_copy(tab_hbm.at[idx_v], rows_v, sem).wait()

            @pl.loop(0, C)
            def biasadd(p):
                for j in range(F_COUNT):
                    r = p * F_COUNT + j
                    for c2 in range(2):
                        sl = pl.ds(c2 * 16, 16)
                        rows_v[r, sl] = rows_v[r, sl] + bias_v[j, sl]

            pltpu.sync_copy(rows_v, out_hbm.at[pl.ds(row0, ROWS)])

    return k(ids_flat, tab_flat, offs_tiled, bias)


def kernel(categorical_ids, tables, feature_embeddings, type_embedding):
    ids_flat = categorical_ids.reshape(N_ROWS)
    tab_flat = tables.reshape(F_COUNT * VOCAB, D)
    bias = feature_embeddings + type_embedding[None, :]
    offs = jnp.arange(F_COUNT, dtype=jnp.int32) * VOCAB
    offs_tiled = jnp.tile(offs, C)
    out = _sc_tokenize(ids_flat, tab_flat, offs_tiled, bias)
    return out.reshape(BATCH, SEQ, F_COUNT, D)


# speedup vs baseline: 2.2241x; 2.2241x over previous
import functools

import jax
import jax.numpy as jnp
from jax import lax
from jax.experimental import pallas as pl
from jax.experimental.pallas import tpu as pltpu
from jax.experimental.pallas import tpu_sc as plsc

F_COUNT = 26
VOCAB = 100000
D = 32
BATCH = 4096
SEQ = 20

N_POS = BATCH * SEQ
NC, NS = 2, 16
NW = NC * NS
POS_PER_W = N_POS // NW
C = 32
CHUNKS = POS_PER_W // C
ROWS = C * F_COUNT
N_ROWS = N_POS * F_COUNT


def _sc_tokenize(ids_flat, tab_flat, offs_tiled, bias):
    mesh = plsc.VectorSubcoreMesh(core_axis_name="c", subcore_axis_name="s")

    @functools.partial(
        pl.kernel,
        mesh=mesh,
        out_type=jax.ShapeDtypeStruct((N_ROWS, D), jnp.float32),
        scratch_types=[
            pltpu.VMEM((ROWS,), jnp.int32),
            pltpu.VMEM((ROWS, D), jnp.float32),
            pltpu.VMEM((ROWS,), jnp.int32),
            pltpu.VMEM((F_COUNT, D), jnp.float32),
            pltpu.SemaphoreType.DMA,
        ],
        compiler_params=pltpu.CompilerParams(use_tc_tiling_on_sc=False),
    )
    def k(ids_hbm, tab_hbm, offs_hbm, bias_hbm, out_hbm,
          idx_v, rows_v, offs_v, bias_v, sem):
        wid = lax.axis_index("s") * NC + lax.axis_index("c")
        base = wid * POS_PER_W * F_COUNT
        pltpu.sync_copy(offs_hbm, offs_v)
        pltpu.sync_copy(bias_hbm, bias_v)

        @pl.loop(0, CHUNKS)
        def chunk(g):
            row0 = base + g * ROWS
            pltpu.sync_copy(ids_hbm.at[pl.ds(row0, ROWS)], idx_v)

            @pl.loop(0, ROWS // 16)
            def addoff(i):
                sl = pl.ds(i * 16, 16)
                idx_v[sl] = idx_v[sl] + offs_v[sl]

            pltpu.async_copy(tab_hbm.at[idx_v], rows_v, sem).wait()

            @pl.loop(0, C)
            def biasadd(p):
                for j in range(F_COUNT):
                    r = p * F_COUNT + j
                    for c2 in range(2):
                        sl = pl.ds(c2 * 16, 16)
                        rows_v[r, sl] = rows_v[r, sl] + bias_v[j, sl]

            pltpu.sync_copy(rows_v, out_hbm.at[pl.ds(row0, ROWS)])

    return k(ids_flat, tab_flat, offs_tiled, bias)


def kernel(categorical_ids, tables, feature_embeddings, type_embedding):
    ids_flat = categorical_ids.reshape(N_ROWS)
    tab_flat = tables.reshape(F_COUNT * VOCAB, D)
    bias = feature_embeddings + type_embedding[None, :]
    offs = jnp.arange(F_COUNT, dtype=jnp.int32) * VOCAB
    offs_tiled = jnp.tile(offs, C)
    out = _sc_tokenize(ids_flat, tab_flat, offs_tiled, bias)
    return out.reshape(BATCH, SEQ, F_COUNT, D)


# speedup vs baseline: 2.7881x; 1.2536x over previous
import functools

import jax
import jax.numpy as jnp
from jax import lax
from jax.experimental import pallas as pl
from jax.experimental.pallas import tpu as pltpu
from jax.experimental.pallas import tpu_sc as plsc

F_COUNT = 26
VOCAB = 100000
D = 32
BATCH = 4096
SEQ = 20

N_POS = BATCH * SEQ
NC, NS = 2, 16
NW = NC * NS
POS_PER_W = N_POS // NW
C = 32
CHUNKS = POS_PER_W // C
ROWS = C * F_COUNT
N_ROWS = N_POS * F_COUNT


def _sc_tokenize(ids_flat, tab_flat, offs_tiled, bias_tiled):
    mesh = plsc.VectorSubcoreMesh(core_axis_name="c", subcore_axis_name="s")

    @functools.partial(
        pl.kernel,
        mesh=mesh,
        out_type=jax.ShapeDtypeStruct((N_ROWS, D), jnp.float32),
        scratch_types=[
            pltpu.VMEM((2, ROWS), jnp.int32),
            pltpu.VMEM((2, ROWS, D), jnp.float32),
            pltpu.VMEM((ROWS,), jnp.int32),
            pltpu.VMEM((ROWS, D), jnp.float32),
            pltpu.SemaphoreType.DMA,
            pltpu.SemaphoreType.DMA,
            pltpu.SemaphoreType.DMA,
        ],
        compiler_params=pltpu.CompilerParams(use_tc_tiling_on_sc=False),
    )
    def k(ids_hbm, tab_hbm, offs_hbm, bias_hbm, out_hbm,
          idx_v, rows_v, offs_v, bias_v, isem, gsem, osem):
        wid = lax.axis_index("s") * NC + lax.axis_index("c")
        base = wid * POS_PER_W * F_COUNT
        pltpu.sync_copy(offs_hbm, offs_v)
        pltpu.sync_copy(bias_hbm, bias_v)

        def addoff(slot):
            @plsc.parallel_loop(0, ROWS // 16, unroll=4)
            def _(i):
                sl = pl.ds(i * 16, 16)
                idx_v[slot, sl] = idx_v[slot, sl] + offs_v[sl]

        def biasadd(slot):
            @plsc.parallel_loop(0, ROWS, unroll=4)
            def _(r):
                for c2 in range(2):
                    sl = pl.ds(c2 * 16, 16)
                    rows_v[slot, r, sl] = rows_v[slot, r, sl] + bias_v[r, sl]

        pltpu.sync_copy(ids_hbm.at[pl.ds(base, ROWS)], idx_v.at[0])
        addoff(0)
        pltpu.async_copy(tab_hbm.at[idx_v.at[0]], rows_v.at[0], gsem)
        pltpu.async_copy(ids_hbm.at[pl.ds(base + ROWS, ROWS)], idx_v.at[1], isem)

        @pl.loop(0, CHUNKS, step=2)
        def outer(g):
            for b in range(2):
                q = g + b
                s, o = b, 1 - b
                row0 = base + q * ROWS

                @pl.when(q + 1 < CHUNKS)
                def _():
                    pltpu.make_async_copy(
                        ids_hbm.at[pl.ds(0, ROWS)], idx_v.at[o], isem).wait()
                    addoff(o)

                @pl.when(q >= 1)
                def _():
                    pltpu.make_async_copy(
                        rows_v.at[o], out_hbm.at[pl.ds(0, ROWS)], osem).wait()

                pltpu.make_async_copy(
                    tab_hbm.at[idx_v.at[s]], rows_v.at[s], gsem).wait()

                @pl.when(q + 1 < CHUNKS)
                def _():
                    pltpu.async_copy(
                        tab_hbm.at[idx_v.at[o]], rows_v.at[o], gsem)

                @pl.when(q + 2 < CHUNKS)
                def _():
                    pltpu.async_copy(
                        ids_hbm.at[pl.ds(row0 + 2 * ROWS, ROWS)],
                        idx_v.at[s], isem)

                biasadd(s)
                pltpu.async_copy(
                    rows_v.at[s], out_hbm.at[pl.ds(row0, ROWS)], osem)

        pltpu.make_async_copy(
            rows_v.at[(CHUNKS - 1) % 2], out_hbm.at[pl.ds(0, ROWS)], osem).wait()

    return k(ids_flat, tab_flat, offs_tiled, bias_tiled)


def kernel(categorical_ids, tables, feature_embeddings, type_embedding):
    ids_flat = categorical_ids.reshape(N_ROWS)
    tab_flat = tables.reshape(F_COUNT * VOCAB, D)
    bias = feature_embeddings + type_embedding[None, :]
    offs = jnp.arange(F_COUNT, dtype=jnp.int32) * VOCAB
    offs_tiled = jnp.tile(offs, C)
    bias_tiled = jnp.tile(bias, (C, 1))
    out = _sc_tokenize(ids_flat, tab_flat, offs_tiled, bias_tiled)
    return out.reshape(BATCH, SEQ, F_COUNT, D)


# speedup vs baseline: 11.7716x; 4.2221x over previous
import functools

import jax
import jax.numpy as jnp
from jax import lax
from jax.experimental import pallas as pl
from jax.experimental.pallas import tpu as pltpu
from jax.experimental.pallas import tpu_sc as plsc

F_COUNT = 26
VOCAB = 100000
D = 32
BATCH = 4096
SEQ = 20

NC, NS = 2, 16
NW = NC * NS
N_PAIRS = F_COUNT * D
PAIRS_PER_W = N_PAIRS // NW
LANES = 16
B_ITERS = BATCH // LANES


def _sc_tokenize(ids_t, tab_t, bias_flat):
    mesh = plsc.VectorSubcoreMesh(core_axis_name="c", subcore_axis_name="s")

    @functools.partial(
        pl.kernel,
        mesh=mesh,
        out_type=jax.ShapeDtypeStruct((SEQ, F_COUNT, D, BATCH), jnp.float32),
        scratch_types=[
            pltpu.VMEM((VOCAB,), jnp.float32),
            pltpu.VMEM((2, BATCH), jnp.int32),
            pltpu.VMEM((2, BATCH), jnp.float32),
            pltpu.VMEM((N_PAIRS * LANES,), jnp.float32),
            pltpu.SemaphoreType.DMA,
            pltpu.SemaphoreType.DMA,
            pltpu.SemaphoreType.DMA,
        ],
        compiler_params=pltpu.CompilerParams(
            use_tc_tiling_on_sc=True, needs_layout_passes=False),
    )
    def k(ids_hbm, tab_hbm, bias_hbm, out_hbm,
          row_v, ids_v, out_v, bias_v, rsem, isem, osem):
        wid = lax.axis_index("s") * NC + lax.axis_index("c")
        p0 = wid * PAIRS_PER_W
        pltpu.sync_copy(bias_hbm, bias_v)

        f0 = p0 // D
        d0 = lax.rem(p0, D)
        pltpu.async_copy(tab_hbm.at[f0, d0, :], row_v, rsem)
        pltpu.async_copy(ids_hbm.at[f0, 0, :], ids_v.at[0], isem)

        @pl.loop(0, PAIRS_PER_W)
        def pair(u):
            p = p0 + u
            f = p // D
            d = lax.rem(p, D)
            bvec = bias_v[pl.ds(p * LANES, LANES)]

            pltpu.make_async_copy(tab_hbm.at[f, d, :], row_v, rsem).wait()

            for l in range(SEQ):
                s = l % 2
                pltpu.make_async_copy(
                    ids_hbm.at[f, 0, :], ids_v.at[s], isem).wait()
                if l + 1 < SEQ:
                    pltpu.async_copy(
                        ids_hbm.at[f, l + 1, :], ids_v.at[1 - s], isem)
                else:
                    @pl.when(u + 1 < PAIRS_PER_W)
                    def _():
                        fn = (p + 1) // D
                        pltpu.async_copy(
                            ids_hbm.at[fn, 0, :], ids_v.at[0], isem)

                @pl.when(u * SEQ + l >= 2)
                def _():
                    pltpu.make_async_copy(
                        out_v.at[s], out_hbm.at[0, 0, 0, :], osem).wait()

                @plsc.parallel_loop(0, B_ITERS, unroll=8)
                def _(i):
                    sl = pl.ds(i * LANES, LANES)
                    out_v[s, sl] = plsc.load_gather(
                        row_v, [ids_v[s, sl]]) + bvec

                pltpu.async_copy(out_v.at[s], out_hbm.at[l, f, d, :], osem)

            @pl.when(u + 1 < PAIRS_PER_W)
            def _():
                fn = (p + 1) // D
                dn = lax.rem(p + 1, D)
                pltpu.async_copy(tab_hbm.at[fn, dn, :], row_v, rsem)

        pltpu.make_async_copy(out_v.at[0], out_hbm.at[0, 0, 0, :], osem).wait()
        pltpu.make_async_copy(out_v.at[1], out_hbm.at[0, 0, 0, :], osem).wait()

    return k(ids_t, tab_t, bias_flat)


def kernel(categorical_ids, tables, feature_embeddings, type_embedding):
    ids_t = jnp.transpose(categorical_ids, (2, 1, 0))
    tab_t = jnp.transpose(tables, (0, 2, 1))
    bias = feature_embeddings + type_embedding[None, :]
    bias_flat = jnp.broadcast_to(
        bias[:, :, None], (F_COUNT, D, LANES)).reshape(-1)
    out_t = _sc_tokenize(ids_t, tab_t, bias_flat)
    return jnp.transpose(out_t, (3, 0, 1, 2))


# speedup vs baseline: 17.7992x; 1.5121x over previous
import functools

import jax
import jax.numpy as jnp
from jax import lax
from jax.experimental import pallas as pl
from jax.experimental.pallas import tpu as pltpu
from jax.experimental.pallas import tpu_sc as plsc

F_COUNT = 26
VOCAB = 100000
D = 32
BATCH = 4096
SEQ = 20

NC, NS = 2, 16
NW = NC * NS
N_PAIRS = F_COUNT * D
PAIRS_PER_W = N_PAIRS // NW
LANES = 16
B_ITERS = BATCH // LANES
ROUNDS = PAIRS_PER_W * SEQ
IRING = 3
ORING = 4


def _sc_tokenize(ids_t, tab_t, bias_flat):
    mesh = plsc.VectorSubcoreMesh(core_axis_name="c", subcore_axis_name="s")

    @functools.partial(
        pl.kernel,
        mesh=mesh,
        out_type=jax.ShapeDtypeStruct((SEQ, F_COUNT, D, BATCH), jnp.float32),
        scratch_types=[
            pltpu.VMEM((VOCAB,), jnp.float32),
            pltpu.VMEM((IRING * BATCH,), jnp.int32),
            pltpu.VMEM((ORING * BATCH,), jnp.float32),
            pltpu.VMEM((PAIRS_PER_W * LANES,), jnp.float32),
            pltpu.SemaphoreType.DMA,
            pltpu.SemaphoreType.DMA,
            pltpu.SemaphoreType.DMA,
        ],
        compiler_params=pltpu.CompilerParams(
            use_tc_tiling_on_sc=True, needs_layout_passes=False),
    )
    def k(ids_hbm, tab_hbm, bias_hbm, out_hbm,
          row_v, ids_v, out_v, bias_v, rsem, isem, osem):
        wid = lax.axis_index("s") * NC + lax.axis_index("c")
        p0 = wid * PAIRS_PER_W
        pltpu.sync_copy(
            bias_hbm.at[pl.ds(p0 * LANES, PAIRS_PER_W * LANES)], bias_v)

        f0 = p0 // D
        d0 = lax.rem(p0, D)
        pltpu.async_copy(tab_hbm.at[f0, d0, :], row_v, rsem)
        pltpu.async_copy(
            ids_hbm.at[f0, 0, :], ids_v.at[pl.ds(0, BATCH)], isem)
        pltpu.async_copy(
            ids_hbm.at[f0, 1, :], ids_v.at[pl.ds(BATCH, BATCH)], isem)

        @pl.loop(0, ROUNDS)
        def round_(q):
            u = q // SEQ
            l = q - u * SEQ
            p = p0 + u
            f = p // D
            d = lax.rem(p, D)
            islot = lax.rem(q, IRING)
            oslot = lax.rem(q, ORING)

            @pl.when(l == 0)
            def _():
                pltpu.make_async_copy(tab_hbm.at[f, d, :], row_v, rsem).wait()

            pltpu.make_async_copy(
                ids_hbm.at[f, 0, :], ids_v.at[pl.ds(0, BATCH)], isem).wait()

            @pl.when(q + 2 < ROUNDS)
            def _():
                q2 = q + 2
                u2 = q2 // SEQ
                l2 = q2 - u2 * SEQ
                f2 = (p0 + u2) // D
                pltpu.async_copy(
                    ids_hbm.at[f2, l2, :],
                    ids_v.at[pl.ds(lax.rem(q2, IRING) * BATCH, BATCH)], isem)

            @pl.when(q >= ORING)
            def _():
                pltpu.make_async_copy(
                    out_v.at[pl.ds(0, BATCH)],
                    out_hbm.at[0, 0, 0, :], osem).wait()

            bvec = bias_v[pl.ds(u * LANES, LANES)]
            ibase = islot * BATCH
            obase = oslot * BATCH

            @plsc.parallel_loop(0, B_ITERS, unroll=8)
            def _(i):
                osl = pl.ds(obase + i * LANES, LANES)
                isl = pl.ds(ibase + i * LANES, LANES)
                out_v[osl] = plsc.load_gather(row_v, [ids_v[isl]]) + bvec

            pltpu.async_copy(
                out_v.at[pl.ds(obase, BATCH)], out_hbm.at[l, f, d, :], osem)

            @pl.when((l == SEQ - 1) & (u + 1 < PAIRS_PER_W))
            def _():
                fn = (p + 1) // D
                dn = lax.rem(p + 1, D)
                pltpu.async_copy(tab_hbm.at[fn, dn, :], row_v, rsem)

        for _ in range(ORING):
            pltpu.make_async_copy(
                out_v.at[pl.ds(0, BATCH)], out_hbm.at[0, 0, 0, :], osem).wait()

    return k(ids_t, tab_t, bias_flat)


def kernel(categorical_ids, tables, feature_embeddings, type_embedding):
    ids_t = jnp.transpose(categorical_ids, (2, 1, 0))
    tab_t = jnp.transpose(tables, (0, 2, 1))
    bias = feature_embeddings + type_embedding[None, :]
    bias_flat = jnp.broadcast_to(
        bias[:, :, None], (F_COUNT, D, LANES)).reshape(-1)
    out_t = _sc_tokenize(ids_t, tab_t, bias_flat)
    return jnp.transpose(out_t, (3, 0, 1, 2))


# speedup vs baseline: 20.1298x; 1.1309x over previous
import functools

import jax
import jax.numpy as jnp
from jax import lax
from jax.experimental import pallas as pl
from jax.experimental.pallas import tpu as pltpu
from jax.experimental.pallas import tpu_sc as plsc

F_COUNT = 26
VOCAB = 100000
D = 32
BATCH = 4096
SEQ = 20

NC, NS = 2, 16
NW = NC * NS
N_PAIRS = F_COUNT * D
PAIRS_PER_W = N_PAIRS // NW
LANES = 16
B_ITERS = BATCH // LANES
ROUNDS = PAIRS_PER_W * SEQ
IRING = 4
ORING = 3
IDEPTH = IRING - 1


def _sc_tokenize(ids_t, tab_t, bias_flat):
    mesh = plsc.VectorSubcoreMesh(core_axis_name="c", subcore_axis_name="s")

    @functools.partial(
        pl.kernel,
        mesh=mesh,
        out_type=jax.ShapeDtypeStruct((SEQ, F_COUNT, D, BATCH), jnp.float32),
        scratch_types=[
            pltpu.VMEM((VOCAB,), jnp.float32),
            pltpu.VMEM((IRING * BATCH,), jnp.int32),
            pltpu.VMEM((ORING * BATCH,), jnp.float32),
            pltpu.VMEM((PAIRS_PER_W * LANES,), jnp.float32),
            pltpu.SemaphoreType.DMA,
            pltpu.SemaphoreType.DMA,
            pltpu.SemaphoreType.DMA,
        ],
        compiler_params=pltpu.CompilerParams(
            use_tc_tiling_on_sc=True, needs_layout_passes=False),
    )
    def k(ids_hbm, tab_hbm, bias_hbm, out_hbm,
          row_v, ids_v, out_v, bias_v, rsem, isem, osem):
        wid = lax.axis_index("s") * NC + lax.axis_index("c")
        p0 = wid * PAIRS_PER_W
        pltpu.sync_copy(
            bias_hbm.at[pl.ds(p0 * LANES, PAIRS_PER_W * LANES)], bias_v)

        f0 = p0 // D
        d0 = lax.rem(p0, D)
        pltpu.async_copy(tab_hbm.at[f0, d0, :], row_v, rsem)
        for j in range(IDEPTH):
            pltpu.async_copy(
                ids_hbm.at[f0, j, :], ids_v.at[pl.ds(j * BATCH, BATCH)], isem)

        @pl.loop(0, ROUNDS)
        def round_(q):
            u = q // SEQ
            l = q - u * SEQ
            p = p0 + u
            f = p // D
            d = lax.rem(p, D)
            islot = lax.rem(q, IRING)
            oslot = lax.rem(q, ORING)

            @pl.when(q + IDEPTH < ROUNDS)
            def _():
                q2 = q + IDEPTH
                u2 = q2 // SEQ
                l2 = q2 - u2 * SEQ
                f2 = (p0 + u2) // D
                pltpu.async_copy(
                    ids_hbm.at[f2, l2, :],
                    ids_v.at[pl.ds(lax.rem(q2, IRING) * BATCH, BATCH)], isem)

            @pl.when(l == 0)
            def _():
                pltpu.make_async_copy(tab_hbm.at[f, d, :], row_v, rsem).wait()

            @pl.when(q >= ORING)
            def _():
                pltpu.make_async_copy(
                    out_v.at[pl.ds(0, BATCH)],
                    out_hbm.at[0, 0, 0, :], osem).wait()

            pltpu.make_async_copy(
                ids_hbm.at[f, 0, :], ids_v.at[pl.ds(0, BATCH)], isem).wait()

            bvec = bias_v[pl.ds(u * LANES, LANES)]
            ibase = islot * BATCH
            obase = oslot * BATCH

            @plsc.parallel_loop(0, B_ITERS, unroll=16)
            def _(i):
                osl = pl.ds(obase + i * LANES, LANES)
                isl = pl.ds(ibase + i * LANES, LANES)
                out_v[osl] = plsc.load_gather(row_v, [ids_v[isl]]) + bvec

            pltpu.async_copy(
                out_v.at[pl.ds(obase, BATCH)], out_hbm.at[l, f, d, :], osem)

            @pl.when((l == SEQ - 1) & (u + 1 < PAIRS_PER_W))
            def _():
                fn = (p + 1) // D
                dn = lax.rem(p + 1, D)
                pltpu.async_copy(tab_hbm.at[fn, dn, :], row_v, rsem)

        for _ in range(ORING):
            pltpu.make_async_copy(
                out_v.at[pl.ds(0, BATCH)], out_hbm.at[0, 0, 0, :], osem).wait()

    return k(ids_t, tab_t, bias_flat)


def kernel(categorical_ids, tables, feature_embeddings, type_embedding):
    ids_t = jnp.transpose(categorical_ids, (2, 1, 0))
    tab_t = jnp.transpose(tables, (0, 2, 1))
    bias = feature_embeddings + type_embedding[None, :]
    bias_flat = jnp.broadcast_to(
        bias[:, :, None], (F_COUNT, D, LANES)).reshape(-1)
    out_t = _sc_tokenize(ids_t, tab_t, bias_flat)
    return jnp.transpose(out_t, (3, 0, 1, 2))


# speedup vs baseline: 20.1929x; 1.0031x over previous
import functools

import jax
import jax.numpy as jnp
from jax import lax
from jax.experimental import pallas as pl
from jax.experimental.pallas import tpu as pltpu
from jax.experimental.pallas import tpu_sc as plsc

F_COUNT = 26
VOCAB = 100000
D = 32
BATCH = 4096
SEQ = 20

NC, NS = 2, 16
NW = NC * NS
N_PAIRS = F_COUNT * D
PAIRS_PER_W = N_PAIRS // NW
LANES = 16
B_ITERS = BATCH // LANES
ROUNDS = PAIRS_PER_W * SEQ
IRING = 4
ORING = 3
IDEPTH = IRING - 1


def _sc_tokenize(ids_t, tab_t, bias_flat):
    mesh = plsc.VectorSubcoreMesh(core_axis_name="c", subcore_axis_name="s")

    @functools.partial(
        pl.kernel,
        mesh=mesh,
        out_type=jax.ShapeDtypeStruct((SEQ, F_COUNT, D, BATCH), jnp.float32),
        scratch_types=[
            pltpu.VMEM((VOCAB,), jnp.float32),
            pltpu.VMEM((IRING * BATCH,), jnp.int32),
            pltpu.VMEM((ORING * BATCH,), jnp.float32),
            pltpu.VMEM((PAIRS_PER_W * LANES,), jnp.float32),
            pltpu.SemaphoreType.DMA,
            pltpu.SemaphoreType.DMA,
            pltpu.SemaphoreType.DMA,
        ],
        compiler_params=pltpu.CompilerParams(
            use_tc_tiling_on_sc=True, needs_layout_passes=False),
    )
    def k(ids_hbm, tab_hbm, bias_hbm, out_hbm,
          row_v, ids_v, out_v, bias_v, rsem, isem, osem):
        wid = lax.axis_index("s") * NC + lax.axis_index("c")
        p0 = wid * PAIRS_PER_W
        pltpu.sync_copy(
            bias_hbm.at[pl.ds(p0 * LANES, PAIRS_PER_W * LANES)], bias_v)

        def row_load(ff, dd):
            pltpu.async_copy(tab_hbm.at[ff, dd, :], row_v, rsem)

        def row_wait(ff, dd):
            pltpu.make_async_copy(tab_hbm.at[ff, dd, :], row_v, rsem).wait()

        f0 = p0 // D
        d0 = lax.rem(p0, D)
        row_load(f0, d0)
        for j in range(IDEPTH):
            pltpu.async_copy(
                ids_hbm.at[f0, j, :], ids_v.at[pl.ds(j * BATCH, BATCH)], isem)

        @pl.loop(0, ROUNDS)
        def round_(q):
            u = q // SEQ
            l = q - u * SEQ
            p = p0 + u
            f = p // D
            d = lax.rem(p, D)
            islot = lax.rem(q, IRING)
            oslot = lax.rem(q, ORING)

            @pl.when(q + IDEPTH < ROUNDS)
            def _():
                q2 = q + IDEPTH
                u2 = q2 // SEQ
                l2 = q2 - u2 * SEQ
                f2 = (p0 + u2) // D
                pltpu.async_copy(
                    ids_hbm.at[f2, l2, :],
                    ids_v.at[pl.ds(lax.rem(q2, IRING) * BATCH, BATCH)], isem)

            @pl.when(l == 0)
            def _():
                row_wait(f, d)

            @pl.when(q >= ORING)
            def _():
                pltpu.make_async_copy(
                    out_v.at[pl.ds(0, BATCH)],
                    out_hbm.at[0, 0, 0, :], osem).wait()

            pltpu.make_async_copy(
                ids_hbm.at[f, 0, :], ids_v.at[pl.ds(0, BATCH)], isem).wait()

            bvec = bias_v[pl.ds(u * LANES, LANES)]
            ibase = islot * BATCH
            obase = oslot * BATCH

            @plsc.parallel_loop(0, B_ITERS, unroll=32)
            def _(i):
                osl = pl.ds(obase + i * LANES, LANES)
                isl = pl.ds(ibase + i * LANES, LANES)
                out_v[osl] = plsc.load_gather(row_v, [ids_v[isl]]) + bvec

            pltpu.async_copy(
                out_v.at[pl.ds(obase, BATCH)], out_hbm.at[l, f, d, :], osem)

            @pl.when((l == SEQ - 1) & (u + 1 < PAIRS_PER_W))
            def _():
                fn = (p + 1) // D
                dn = lax.rem(p + 1, D)
                row_load(fn, dn)

        for _ in range(ORING):
            pltpu.make_async_copy(
                out_v.at[pl.ds(0, BATCH)], out_hbm.at[0, 0, 0, :], osem).wait()

    return k(ids_t, tab_t, bias_flat)


def kernel(categorical_ids, tables, feature_embeddings, type_embedding):
    ids_t = jnp.transpose(categorical_ids, (2, 1, 0))
    tab_t = jnp.transpose(tables, (0, 2, 1))
    bias = feature_embeddings + type_embedding[None, :]
    bias_flat = jnp.broadcast_to(
        bias[:, :, None], (F_COUNT, D, LANES)).reshape(-1)
    out_t = _sc_tokenize(ids_t, tab_t, bias_flat)
    return jnp.transpose(out_t, (3, 0, 1, 2))


# speedup vs baseline: 20.5604x; 1.0182x over previous
import functools

import jax
import jax.numpy as jnp
from jax import lax
from jax.experimental import pallas as pl
from jax.experimental.pallas import tpu as pltpu
from jax.experimental.pallas import tpu_sc as plsc

F_COUNT = 26
VOCAB = 100000
D = 32
BATCH = 4096
SEQ = 20

NC, NS = 2, 16
NW = NC * NS
N_PAIRS = F_COUNT * D
PAIRS_PER_W = N_PAIRS // NW
LANES = 16
B_ITERS = BATCH // LANES
ROUNDS = PAIRS_PER_W * SEQ
IRING = 5
ORING = 2
IDEPTH = IRING - 1


def _sc_tokenize(ids_t, tab_t, bias_flat):
    mesh = plsc.VectorSubcoreMesh(core_axis_name="c", subcore_axis_name="s")

    @functools.partial(
        pl.kernel,
        mesh=mesh,
        out_type=jax.ShapeDtypeStruct((SEQ, F_COUNT, D, BATCH), jnp.float32),
        scratch_types=[
            pltpu.VMEM((VOCAB,), jnp.float32),
            pltpu.VMEM((IRING * BATCH,), jnp.int32),
            pltpu.VMEM((ORING * BATCH,), jnp.float32),
            pltpu.VMEM((PAIRS_PER_W * LANES,), jnp.float32),
            pltpu.SemaphoreType.DMA,
            pltpu.SemaphoreType.DMA,
            pltpu.SemaphoreType.DMA,
        ],
        compiler_params=pltpu.CompilerParams(
            use_tc_tiling_on_sc=True, needs_layout_passes=False),
    )
    def k(ids_hbm, tab_hbm, bias_hbm, out_hbm,
          row_v, ids_v, out_v, bias_v, rsem, isem, osem):
        wid = lax.axis_index("s") * NC + lax.axis_index("c")
        p0 = wid * PAIRS_PER_W
        pltpu.sync_copy(
            bias_hbm.at[pl.ds(p0 * LANES, PAIRS_PER_W * LANES)], bias_v)

        def row_load(ff, dd):
            pltpu.async_copy(tab_hbm.at[ff, dd, :], row_v, rsem)

        def row_wait(ff, dd):
            pltpu.make_async_copy(tab_hbm.at[ff, dd, :], row_v, rsem).wait()

        f0 = p0 // D
        d0 = lax.rem(p0, D)
        row_load(f0, d0)
        for j in range(IDEPTH):
            pltpu.async_copy(
                ids_hbm.at[f0, j, :], ids_v.at[pl.ds(j * BATCH, BATCH)], isem)

        @pl.loop(0, ROUNDS)
        def round_(q):
            u = q // SEQ
            l = q - u * SEQ
            p = p0 + u
            f = p // D
            d = lax.rem(p, D)
            islot = lax.rem(q, IRING)
            oslot = lax.rem(q, ORING)

            @pl.when(q + IDEPTH < ROUNDS)
            def _():
                q2 = q + IDEPTH
                u2 = q2 // SEQ
                l2 = q2 - u2 * SEQ
                f2 = (p0 + u2) // D
                pltpu.async_copy(
                    ids_hbm.at[f2, l2, :],
                    ids_v.at[pl.ds(lax.rem(q2, IRING) * BATCH, BATCH)], isem)

            @pl.when(l == 0)
            def _():
                row_wait(f, d)

            @pl.when(q >= ORING)
            def _():
                pltpu.make_async_copy(
                    out_v.at[pl.ds(0, BATCH)],
                    out_hbm.at[0, 0, 0, :], osem).wait()

            pltpu.make_async_copy(
                ids_hbm.at[f, 0, :], ids_v.at[pl.ds(0, BATCH)], isem).wait()

            bvec = bias_v[pl.ds(u * LANES, LANES)]
            ibase = islot * BATCH
            obase = oslot * BATCH

            @plsc.parallel_loop(0, B_ITERS, unroll=32)
            def _(i):
                osl = pl.ds(obase + i * LANES, LANES)
                isl = pl.ds(ibase + i * LANES, LANES)
                out_v[osl] = plsc.load_gather(row_v, [ids_v[isl]]) + bvec

            pltpu.async_copy(
                out_v.at[pl.ds(obase, BATCH)], out_hbm.at[l, f, d, :], osem)

            @pl.when((l == SEQ - 1) & (u + 1 < PAIRS_PER_W))
            def _():
                fn = (p + 1) // D
                dn = lax.rem(p + 1, D)
                row_load(fn, dn)

        for _ in range(ORING):
            pltpu.make_async_copy(
                out_v.at[pl.ds(0, BATCH)], out_hbm.at[0, 0, 0, :], osem).wait()

    return k(ids_t, tab_t, bias_flat)


def kernel(categorical_ids, tables, feature_embeddings, type_embedding):
    ids_t = jnp.transpose(categorical_ids, (2, 1, 0))
    tab_t = jnp.transpose(tables, (0, 2, 1))
    bias = feature_embeddings + type_embedding[None, :]
    bias_flat = jnp.broadcast_to(
        bias[:, :, None], (F_COUNT, D, LANES)).reshape(-1)
    out_t = _sc_tokenize(ids_t, tab_t, bias_flat)
    return jnp.transpose(out_t, (3, 0, 1, 2))
